# baseline (speedup 1.0000x reference)
import jax
import jax.numpy as jnp
from jax.experimental import pallas as pl
from jax.experimental.pallas import tpu as pltpu

_Q, _K, _D = 8, 1024, 32
_TILE = 2048


def _rvq_body(x_ref, cb_ref, quant_ref, codes_ref, loss_ref):
    i = pl.program_id(0)

    @pl.when(i == 0)
    def _init():
        loss_ref[...] = jnp.zeros_like(loss_ref)

    residual = x_ref[...]
    quant = jnp.zeros_like(residual)
    m = residual.shape[0]
    iota = jax.lax.broadcasted_iota(jnp.int32, (m, _K), 1)
    for q in range(_Q):
        cb = cb_ref[q]
        rsq = jnp.sum(residual * residual, axis=1, keepdims=True)
        csq = jnp.sum(cb * cb, axis=1)
        dot = jax.lax.dot_general(
            residual.astype(jnp.bfloat16), cb.astype(jnp.bfloat16),
            (((1,), (1,)), ((), ())),
            preferred_element_type=jnp.float32)
        dist = rsq + csq[None, :] - 2.0 * dot
        mind = jnp.min(dist, axis=1, keepdims=True)
        idx = jnp.min(jnp.where(dist == mind, iota, _K), axis=1)
        codes_ref[q, :] = idx
        one_hot = (iota == idx[:, None]).astype(jnp.float32)
        embeds = jax.lax.dot_general(
            one_hot, cb, (((1,), (0,)), ((), ())),
            precision=jax.lax.Precision.HIGHEST,
            preferred_element_type=jnp.float32)
        quant = quant + embeds
        residual = residual - embeds
        diff = residual - embeds
        loss_ref[q, :] += jnp.sum(diff * diff)
    quant_ref[...] = quant


def kernel(x, codebooks):
    b, t, d = x.shape
    n = b * t
    xf = x.reshape(n, d)
    grid = (n // _TILE,)
    quant, codes, loss_rows = pl.pallas_call(
        _rvq_body,
        grid=grid,
        in_specs=[
            pl.BlockSpec((_TILE, d), lambda i: (i, 0)),
            pl.BlockSpec((_Q, _K, _D), lambda i: (0, 0, 0)),
        ],
        out_specs=[
            pl.BlockSpec((_TILE, d), lambda i: (i, 0)),
            pl.BlockSpec((_Q, _TILE), lambda i: (0, i)),
            pl.BlockSpec((_Q, 128), lambda i: (0, 0)),
        ],
        out_shape=[
            jax.ShapeDtypeStruct((n, d), jnp.float32),
            jax.ShapeDtypeStruct((_Q, n), jnp.int32),
            jax.ShapeDtypeStruct((_Q, 128), jnp.float32),
        ],
        compiler_params=pltpu.CompilerParams(
            dimension_semantics=("arbitrary",),
        ),
    )(xf, codebooks)
    quantized = quant.reshape(b, t, d)
    codes_out = codes.reshape(_Q, b, t)
    loss = jnp.sum(loss_rows[:, 0] / jnp.float32(n * d)) * jnp.float32(0.25)
    return (quantized, loss, codes_out)


# speedup vs baseline: 1.5509x; 1.5509x over previous
import jax
import jax.numpy as jnp
from jax.experimental import pallas as pl
from jax.experimental.pallas import tpu as pltpu

_Q, _K, _D = 8, 1024, 32
_TILE = 2048


def _rvq_body(x_ref, cb_ref, quant_ref, codes_ref, loss_ref):
    i = pl.program_id(0)

    @pl.when(i == 0)
    def _init():
        loss_ref[...] = jnp.zeros_like(loss_ref)

    residual = x_ref[...]
    quant = jnp.zeros_like(residual)
    m = residual.shape[0]
    iota = jax.lax.broadcasted_iota(jnp.int32, (m, _K), 1)
    for q in range(_Q):
        cb = cb_ref[q]
        rsq = jnp.sum(residual * residual, axis=1, keepdims=True)
        csq = jnp.sum(cb * cb, axis=1)
        rm2 = (-2.0 * residual).astype(jnp.bfloat16)
        dotm2 = jax.lax.dot_general(
            rm2, cb.astype(jnp.bfloat16), (((1,), (1,)), ((), ())),
            preferred_element_type=jnp.float32)
        dist = (rsq + csq[None, :]) + dotm2
        idx = jnp.argmin(dist, axis=1).astype(jnp.int32)
        codes_ref[:, q] = idx
        one_hot = (iota == idx[:, None]).astype(jnp.bfloat16)
        cb_h = cb.astype(jnp.bfloat16)
        r1 = cb - cb_h.astype(jnp.float32)
        cb_m = r1.astype(jnp.bfloat16)
        cb_l = (r1 - cb_m.astype(jnp.float32)).astype(jnp.bfloat16)
        dims = (((1,), (0,)), ((), ()))
        embeds = ((jax.lax.dot_general(one_hot, cb_h, dims,
                                       preferred_element_type=jnp.float32)
                   + jax.lax.dot_general(one_hot, cb_m, dims,
                                         preferred_element_type=jnp.float32))
                  + jax.lax.dot_general(one_hot, cb_l, dims,
                                        preferred_element_type=jnp.float32))
        quant = quant + embeds
        residual = residual - embeds
        diff = residual - embeds
        loss_ref[q, :] += jnp.sum(diff * diff)
    quant_ref[...] = quant


def kernel(x, codebooks):
    b, t, d = x.shape
    n = b * t
    xf = x.reshape(n, d)
    grid = (n // _TILE,)
    quant, codes, loss_rows = pl.pallas_call(
        _rvq_body,
        grid=grid,
        in_specs=[
            pl.BlockSpec((_TILE, d), lambda i: (i, 0)),
            pl.BlockSpec((_Q, _K, _D), lambda i: (0, 0, 0)),
        ],
        out_specs=[
            pl.BlockSpec((_TILE, d), lambda i: (i, 0)),
            pl.BlockSpec((_TILE, _Q), lambda i: (i, 0)),
            pl.BlockSpec((_Q, 128), lambda i: (0, 0)),
        ],
        out_shape=[
            jax.ShapeDtypeStruct((n, d), jnp.float32),
            jax.ShapeDtypeStruct((n, _Q), jnp.int32),
            jax.ShapeDtypeStruct((_Q, 128), jnp.float32),
        ],
        compiler_params=pltpu.CompilerParams(
            dimension_semantics=("arbitrary",),
        ),
    )(xf, codebooks)
    quantized = quant.reshape(b, t, d)
    codes_out = codes.T.reshape(_Q, b, t)
    loss = jnp.sum(loss_rows[:, 0] / jnp.float32(n * d)) * jnp.float32(0.25)
    return (quantized, loss, codes_out)
